# baseline (speedup 1.0000x reference)
import functools

import jax
import jax.numpy as jnp
from jax import lax
from jax.experimental import pallas as pl
from jax.experimental.pallas import tpu as pltpu
from jax.experimental.pallas import tpu_sc as plsc

B = 16384
D = 64

_info = plsc.get_sparse_core_info()
_NC, _NS = _info.num_cores, _info.num_subcores
_NW = _NC * _NS
_BPW = B // _NW
_CHUNK = 128
_NCHUNK = _BPW // _CHUNK


def _dot_kernel(user_idx, content_idx, user_t2, content_t2,
                out_hbm, idx_v, tidx_v, col_v, ublk_v, cblk_v, out_v, sem):
    wid = lax.axis_index("s") * _NC + lax.axis_index("c")
    base = wid * _BPW

    pltpu.sync_copy(user_idx.at[pl.ds(wid * 4, 4)], idx_v.at[pl.ds(0, 4)])
    pltpu.sync_copy(content_idx.at[pl.ds(wid * 4, 4)], idx_v.at[pl.ds(4, 4)])

    def split(i, carry):
        v = idx_v[i // 8, pl.ds((i % 8) * 16, 16)]
        tidx_v[i // 8, pl.ds((i % 8) * 16, 16)] = lax.shift_right_logical(v, 1)
        col_v[pl.ds(i * 16, 16)] = lax.shift_left(lax.bitwise_and(v, 1), 6)
        return carry

    lax.fori_loop(0, 2 * _BPW // 16, split, 0)

    lanes = lax.iota(jnp.int32, 16)

    def chunk(c, carry):
        cp_u = pltpu.async_copy(user_t2.at[tidx_v.at[c]], ublk_v, sem)
        cp_c = pltpu.async_copy(
            content_t2.at[tidx_v.at[_NCHUNK + c]], cblk_v, sem)
        cp_u.wait()
        cp_c.wait()

        def block(h, carry2):
            i = c * _CHUNK + h * 16
            tloc = lanes + h * 16
            ucol = col_v[pl.ds(i, 16)]
            ccol = col_v[pl.ds(_BPW + i, 16)]
            acc0 = jnp.zeros((16,), jnp.float32)
            acc1 = jnp.zeros((16,), jnp.float32)
            for d in range(0, D, 2):
                acc0 += (plsc.load_gather(ublk_v, [tloc, ucol + d])
                         * plsc.load_gather(cblk_v, [tloc, ccol + d]))
                acc1 += (plsc.load_gather(ublk_v, [tloc, ucol + (d + 1)])
                         * plsc.load_gather(cblk_v, [tloc, ccol + (d + 1)]))
            out_v[pl.ds(i, 16)] = acc0 + acc1
            return carry2

        lax.fori_loop(0, _CHUNK // 16, block, 0)
        return carry

    lax.fori_loop(0, _NCHUNK, chunk, 0)

    pltpu.sync_copy(out_v, out_hbm.at[pl.ds(base, _BPW)])


@jax.jit
def _run(user_idx2d, content_idx2d, user_t2, content_t2):
    mesh = plsc.VectorSubcoreMesh(core_axis_name="c", subcore_axis_name="s")
    f = functools.partial(
        pl.kernel, mesh=mesh,
        out_type=jax.ShapeDtypeStruct((B,), jnp.float32),
        compiler_params=pltpu.CompilerParams(needs_layout_passes=False),
        scratch_types=[
            pltpu.VMEM((8, 128), jnp.int32),
            pltpu.VMEM((8, 128), jnp.int32),
            pltpu.VMEM((2 * _BPW,), jnp.int32),
            pltpu.VMEM((_CHUNK, 128), jnp.float32),
            pltpu.VMEM((_CHUNK, 128), jnp.float32),
            pltpu.VMEM((_BPW,), jnp.float32),
            pltpu.SemaphoreType.DMA,
        ],
    )(_dot_kernel)
    return f(user_idx2d, content_idx2d, user_t2, content_t2)


def kernel(user, content, user_table, content_table):
    nu = user_table.shape[0]
    nc = content_table.shape[0]
    out = _run(user.reshape(B // 128, 128),
               content.reshape(B // 128, 128),
               user_table.reshape(nu // 2, 2 * D),
               content_table.reshape(nc // 2, 2 * D))
    return out.reshape(B, 1)


# speedup vs baseline: 1.3612x; 1.3612x over previous
import functools

import jax
import jax.numpy as jnp
from jax import lax
from jax.experimental import pallas as pl
from jax.experimental.pallas import tpu as pltpu
from jax.experimental.pallas import tpu_sc as plsc

B = 16384
D = 64
NU = 1000000
NC_TAB = 100000

_info = plsc.get_sparse_core_info()
_NCORE, _NSUB = _info.num_cores, _info.num_subcores
_NW = _NCORE * _NSUB
_BPW = B // _NW

_U_LAST = (NU - 1) // 128
_C_LAST = (NC_TAB - 1) // 128


def _sread(ref, i):
    return ref[pl.ds(i, 16)][0]


def _stage_segments(idx_v, seg_s):
    seg_s[0] = 0

    def body(p, ns):
        new = lax.ne(lax.shift_right_logical(_sread(idx_v, p), 7),
                     lax.shift_right_logical(_sread(idx_v, p - 1), 7))

        def write(n):
            seg_s[n] = p
            return n + 1

        return lax.cond(new, write, lambda n: n, ns)

    ns = lax.fori_loop(1, _BPW, body, 1)
    seg_s[ns] = _BPW
    return ns


def _fetch_slab(table_tt, buf, parity, slab, last, sem, *, sync):
    @pl.when(lax.ne(slab, last))
    def _():
        off = pl.multiple_of(slab * 128, 128)
        cp = pltpu.async_copy(table_tt.at[:, pl.ds(off, 128)],
                              buf.at[parity], sem)
        if sync:
            cp.wait()


def _wait_slab(table_tt, buf, slab, last, sem):
    @pl.when(lax.ne(slab, last))
    def _():
        pltpu.make_async_copy(table_tt.at[:, pl.ds(0, 128)],
                              buf.at[0], sem).wait()


def _extract_cols(buf, parity, col, lanes, stage, stage_off):
    par = jnp.full((16,), parity, jnp.int32)
    cols = jnp.full((16,), col, jnp.int32)
    for k in range(4):
        v = plsc.load_gather(buf, [par, lanes + k * 16, cols])
        stage[pl.ds(stage_off + k * 16, 16)] = v


def _walk(idx_v, seg_s, ns, table_tt, buf, last, tail_base, slabsem,
          per_position):
    slab0 = lax.shift_right_logical(_sread(idx_v, 0), 7)
    _fetch_slab(table_tt, buf, 0, slab0, last, slabsem, sync=True)

    def prefetch(seg_next):
        @pl.when(seg_next < ns)
        def _():
            st = seg_s[jnp.minimum(seg_next, _BPW)]
            nslab = lax.shift_right_logical(_sread(idx_v, st), 7)
            _fetch_slab(table_tt, buf, seg_next % 2, nslab, last, slabsem,
                        sync=False)

    prefetch(jnp.int32(1))

    def body(p, carry):
        seg, nextstart = carry

        adv = lax.eq(p, nextstart)
        seg = lax.select(adv, seg + 1, seg)
        nextstart = lax.select(adv, seg_s[jnp.minimum(seg + 1, _BPW)],
                               nextstart)

        @pl.when(adv)
        def _():
            slab = lax.shift_right_logical(_sread(idx_v, p), 7)
            _wait_slab(table_tt, buf, slab, last, slabsem)
            prefetch(seg + 1)

        r = _sread(idx_v, p)
        slab_p = lax.shift_right_logical(r, 7)
        is_last = lax.eq(slab_p, last)
        parity = lax.select(is_last, jnp.int32(2), seg % 2)
        col = lax.select(is_last, r - tail_base, lax.bitwise_and(r, 127))
        per_position(p, parity, col)
        return seg, nextstart

    lax.fori_loop(0, _BPW, body, (jnp.int32(0), seg_s[1]))


def _user_kernel(user_sorted, user_tt, user_tail, emb_out,
                 seg_scratch, idx_v, stage, buf, slabsem):
    wid = lax.axis_index("s") * _NCORE + lax.axis_index("c")
    base = wid * _BPW

    pltpu.sync_copy(user_sorted.at[pl.ds(base, _BPW)],
                    idx_v.at[pl.ds(0, _BPW)])
    pltpu.sync_copy(user_tail, buf.at[2])
    ns = _stage_segments(idx_v, seg_scratch)

    lanes = lax.iota(jnp.int32, 16)

    def per_position(p, parity, col):
        _extract_cols(buf, parity, col, lanes, stage, p * D)

    _walk(idx_v, seg_scratch, ns, user_tt, buf, _U_LAST, NU - 128,
          slabsem, per_position)

    pltpu.sync_copy(stage, emb_out.at[pl.ds(base * D, _BPW * D)])


def _content_kernel(content_sorted, comp_perm, content_tt, content_tail,
                    u_emb, out_hbm,
                    seg_scratch, idx_v, comp_v, cstage, urows, buf,
                    out_v, slabsem, usem):
    wid = lax.axis_index("s") * _NCORE + lax.axis_index("c")
    base = wid * _BPW

    pltpu.sync_copy(content_sorted.at[pl.ds(base, _BPW)],
                    idx_v.at[pl.ds(0, _BPW)])
    pltpu.sync_copy(comp_perm.at[pl.ds(base, _BPW)],
                    comp_v.at[pl.ds(0, _BPW)])
    pltpu.sync_copy(content_tail, buf.at[2])
    ns = _stage_segments(idx_v, seg_scratch)

    lanes = lax.iota(jnp.int32, 16)

    def per_position(p, parity, col):
        _extract_cols(buf, parity, col, lanes, cstage, (p % 32) * D)
        up = _sread(comp_v, p)
        pltpu.async_copy(u_emb.at[pl.ds(up * D, D)],
                         urows.at[pl.ds((p % 32) * D, D)], usem)

        @pl.when(lax.eq(p % 16, 15))
        def _():
            pltpu.make_async_copy(u_emb.at[pl.ds(0, 16 * D)],
                                  urows.at[pl.ds(0, 16 * D)], usem).wait()
            blk = ((p - 15) % 32)
            acc0 = jnp.zeros((16,), jnp.float32)
            acc1 = jnp.zeros((16,), jnp.float32)
            rows = lanes * D + blk * D
            for d in range(0, D, 2):
                acc0 += (plsc.load_gather(urows, [rows + d])
                         * plsc.load_gather(cstage, [rows + d]))
                acc1 += (plsc.load_gather(urows, [rows + (d + 1)])
                         * plsc.load_gather(cstage, [rows + (d + 1)]))
            out_v[pl.ds(p - 15, 16)] = acc0 + acc1

    _walk(idx_v, seg_scratch, ns, content_tt, buf, _C_LAST, NC_TAB - 128,
          slabsem, per_position)

    pltpu.sync_copy(out_v, out_hbm.at[pl.ds(base, _BPW)])


@jax.jit
def _run(user_sorted, content_sorted, comp_perm, user_tt, content_tt,
         user_tail, content_tail):
    mesh = plsc.VectorSubcoreMesh(core_axis_name="c", subcore_axis_name="s")
    params = pltpu.CompilerParams(needs_layout_passes=False)

    k1 = functools.partial(
        pl.kernel, mesh=mesh,
        out_type=jax.ShapeDtypeStruct((B * D,), jnp.float32),
        compiler_params=params,
        scratch_types=[
            pltpu.SMEM((_BPW + 1,), jnp.int32),
            pltpu.VMEM((_BPW + 16,), jnp.int32),
            pltpu.VMEM((_BPW * D,), jnp.float32),
            pltpu.VMEM((3, D, 128), jnp.float32),
            pltpu.SemaphoreType.DMA,
        ],
    )(_user_kernel)
    u_emb = k1(user_sorted, user_tt, user_tail)

    k2 = functools.partial(
        pl.kernel, mesh=mesh,
        out_type=jax.ShapeDtypeStruct((B,), jnp.float32),
        compiler_params=params,
        scratch_types=[
            pltpu.SMEM((_BPW + 1,), jnp.int32),
            pltpu.VMEM((_BPW + 16,), jnp.int32),
            pltpu.VMEM((_BPW + 16,), jnp.int32),
            pltpu.VMEM((32 * D,), jnp.float32),
            pltpu.VMEM((32 * D,), jnp.float32),
            pltpu.VMEM((3, D, 128), jnp.float32),
            pltpu.VMEM((_BPW,), jnp.float32),
            pltpu.SemaphoreType.DMA,
            pltpu.SemaphoreType.DMA,
        ],
    )(_content_kernel)
    return k2(content_sorted, comp_perm, content_tt, content_tail, u_emb)


def kernel(user, content, user_table, content_table):
    u_order = jnp.argsort(user)
    c_order = jnp.argsort(content)
    user_sorted = user[u_order]
    content_sorted = content[c_order]
    inv_u = jnp.zeros((B,), jnp.int32).at[u_order].set(
        jnp.arange(B, dtype=jnp.int32))
    comp = inv_u[c_order]
    out_sorted = _run(user_sorted, content_sorted, comp,
                      user_table.T, content_table.T,
                      user_table[NU - 128:, :].T,
                      content_table[NC_TAB - 128:, :].T)
    out = jnp.zeros((B,), jnp.float32).at[c_order].set(out_sorted)
    return out.reshape(B, 1)


# speedup vs baseline: 1.8580x; 1.3650x over previous
import functools

import jax
import jax.numpy as jnp
from jax import lax
from jax.experimental import pallas as pl
from jax.experimental.pallas import tpu as pltpu
from jax.experimental.pallas import tpu_sc as plsc

B = 16384
D = 64
NU = 1000000
NC_TAB = 100000

_info = plsc.get_sparse_core_info()
_NCORE, _NSUB = _info.num_cores, _info.num_subcores
_NW = _NCORE * _NSUB
_BPW = B // _NW
_FP = 16

_U_LAST = (NU - 1) // 128
_C_LAST = (NC_TAB - 1) // 128


def _sread(ref, i):
    return ref[pl.ds(i, 16)][0]


def _stage_segments(idx_v, seg_s):
    seg_s[0] = 0

    def body(p, ns):
        new = lax.ne(lax.shift_right_logical(_sread(idx_v, _FP + p), 7),
                     lax.shift_right_logical(_sread(idx_v, _FP + p - 1), 7))

        def write(n):
            seg_s[n] = p
            return n + 1

        return lax.cond(new, write, lambda n: n, ns)

    ns = lax.fori_loop(1, _BPW, body, 1)
    seg_s[ns] = _BPW
    return ns


def _stage_slots(idx_v, slot_v, col_v, grp, last, tail_base):
    lastv = jnp.full((16,), last, jnp.int32)

    def body(k, segc):
        v = idx_v[pl.ds(_FP + k * 16, 16)]
        prev = idx_v[pl.ds(_FP + k * 16 - 1, 16)]
        vs = lax.shift_right_logical(v, 7)
        b = (vs != lax.shift_right_logical(prev, 7)).astype(jnp.int32)
        segs = plsc.cumsum(b) + segc
        is_last = vs == lastv
        slot = jnp.where(is_last, jnp.int32(grp),
                         lax.bitwise_and(segs, grp - 1))
        col = jnp.where(is_last, v - tail_base, lax.bitwise_and(v, 127))
        slot_v[pl.ds(k * 16, 16)] = slot
        col_v[pl.ds(k * 16, 16)] = col
        return segs[15]

    lax.fori_loop(0, _BPW // 16, body, jnp.int32(0))


def _walk_extract(idx_v, seg_s, ns, slot_v, col_v, table_tt, buf, stage,
                  grp, last, slabsem):
    lanes = lax.iota(jnp.int32, 16)
    ngrp = (ns + grp - 1) // grp

    def group(g, carry):
        nf = jnp.int32(0)
        for j in range(grp):
            s = g * grp + j
            valid = s < ns
            st = seg_s[jnp.minimum(s, ns)]
            slab = lax.shift_right_logical(_sread(idx_v, _FP + st), 7)
            fire = jnp.logical_and(valid, lax.ne(slab, last))

            @pl.when(fire)
            def _():
                off = pl.multiple_of(slab * 128, 128)
                pltpu.async_copy(table_tt.at[:, pl.ds(off, 128)],
                                 buf.at[j], slabsem)

            nf = nf + fire.astype(jnp.int32)

        def drain(i, c):
            pltpu.make_async_copy(table_tt.at[:, pl.ds(0, 128)],
                                  buf.at[0], slabsem).wait()
            return c

        lax.fori_loop(0, nf, drain, 0)

        pstart = seg_s[jnp.minimum(g * grp, ns)]
        pend = seg_s[jnp.minimum((g + 1) * grp, ns)]

        def pblock(pb, c):
            pos = pb * 16 + lanes
            m = jnp.logical_and(pos >= pstart, pos < pend)
            slots = slot_v[pl.ds(pb * 16, 16)]
            cols = col_v[pl.ds(pb * 16, 16)]
            wbase = pos * D
            for d in range(D):
                dv = jnp.full((16,), d, jnp.int32)
                val = plsc.load_gather(buf, [slots, dv, cols])
                plsc.store_scatter(stage, [wbase + d], val, mask=m)
            return c

        lax.fori_loop(lax.shift_right_logical(pstart, 4),
                      lax.shift_right_logical(pend + 15, 4), pblock, 0)
        return carry

    lax.fori_loop(0, ngrp, group, 0)


def _stage_idx(src, base, idx_v):
    pltpu.sync_copy(src.at[pl.ds(base, _BPW)], idx_v.at[pl.ds(_FP, _BPW)])
    first = idx_v[pl.ds(_FP, 16)]
    idx_v[pl.ds(0, 16)] = jnp.full((16,), 1, jnp.int32) * first[0]


_UGRP = 8
_CGRP = 4


def _user_kernel(user_sorted, user_tt, user_tail, emb_out,
                 seg_s, idx_v, slot_v, col_v, stage, buf, slabsem):
    wid = lax.axis_index("s") * _NCORE + lax.axis_index("c")
    base = wid * _BPW

    _stage_idx(user_sorted, base, idx_v)
    pltpu.sync_copy(user_tail, buf.at[_UGRP])
    ns = _stage_segments(idx_v, seg_s)
    _stage_slots(idx_v, slot_v, col_v, _UGRP, _U_LAST, NU - 128)
    _walk_extract(idx_v, seg_s, ns, slot_v, col_v, user_tt, buf, stage,
                  _UGRP, _U_LAST, slabsem)
    pltpu.sync_copy(stage, emb_out.at[pl.ds(base * D, _BPW * D)])


def _content_kernel(content_sorted, comp_perm, content_tt, content_tail,
                    u_emb, out_hbm,
                    seg_s, idx_v, comp_v, slot_v, col_v, stage, urows,
                    buf, out_v, slabsem, usem):
    wid = lax.axis_index("s") * _NCORE + lax.axis_index("c")
    base = wid * _BPW

    _stage_idx(content_sorted, base, idx_v)
    pltpu.sync_copy(comp_perm.at[pl.ds(base, _BPW)],
                    comp_v.at[pl.ds(0, _BPW)])
    pltpu.sync_copy(content_tail, buf.at[_CGRP])
    ns = _stage_segments(idx_v, seg_s)
    _stage_slots(idx_v, slot_v, col_v, _CGRP, _C_LAST, NC_TAB - 128)

    def firep(p, c):
        up = _sread(comp_v, p)
        pltpu.async_copy(u_emb.at[pl.ds(pl.multiple_of(up * D, D), D)],
                         urows.at[pl.ds(pl.multiple_of(p * D, D), D)],
                         usem)
        return c

    lax.fori_loop(0, _BPW, firep, 0)

    _walk_extract(idx_v, seg_s, ns, slot_v, col_v, content_tt, buf, stage,
                  _CGRP, _C_LAST, slabsem)

    pltpu.make_async_copy(u_emb.at[pl.ds(0, _BPW * D)], urows, usem).wait()

    lanes = lax.iota(jnp.int32, 16)

    def dblk(h, c):
        rows = (lanes + h * 16) * D
        acc0 = jnp.zeros((16,), jnp.float32)
        acc1 = jnp.zeros((16,), jnp.float32)
        for d in range(0, D, 2):
            acc0 += (plsc.load_gather(urows, [rows + d])
                     * plsc.load_gather(stage, [rows + d]))
            acc1 += (plsc.load_gather(urows, [rows + (d + 1)])
                     * plsc.load_gather(stage, [rows + (d + 1)]))
        out_v[pl.ds(h * 16, 16)] = acc0 + acc1
        return c

    lax.fori_loop(0, _BPW // 16, dblk, 0)

    pltpu.sync_copy(out_v, out_hbm.at[pl.ds(base, _BPW)])


@jax.jit
def _run(user_sorted, content_sorted, comp_perm, user_tt, content_tt,
         user_tail, content_tail):
    mesh = plsc.VectorSubcoreMesh(core_axis_name="c", subcore_axis_name="s")
    params = pltpu.CompilerParams(needs_layout_passes=False)

    k1 = functools.partial(
        pl.kernel, mesh=mesh,
        out_type=jax.ShapeDtypeStruct((B * D,), jnp.float32),
        compiler_params=params,
        scratch_types=[
            pltpu.SMEM((_BPW + 1,), jnp.int32),
            pltpu.VMEM((_FP + _BPW + 16,), jnp.int32),
            pltpu.VMEM((_BPW + 16,), jnp.int32),
            pltpu.VMEM((_BPW + 16,), jnp.int32),
            pltpu.VMEM((_BPW * D,), jnp.float32),
            pltpu.VMEM((_UGRP + 1, D, 128), jnp.float32),
            pltpu.SemaphoreType.DMA,
        ],
    )(_user_kernel)
    u_emb = k1(user_sorted, user_tt, user_tail)

    k2 = functools.partial(
        pl.kernel, mesh=mesh,
        out_type=jax.ShapeDtypeStruct((B,), jnp.float32),
        compiler_params=params,
        scratch_types=[
            pltpu.SMEM((_BPW + 1,), jnp.int32),
            pltpu.VMEM((_FP + _BPW + 16,), jnp.int32),
            pltpu.VMEM((_BPW + 16,), jnp.int32),
            pltpu.VMEM((_BPW + 16,), jnp.int32),
            pltpu.VMEM((_BPW + 16,), jnp.int32),
            pltpu.VMEM((_BPW * D,), jnp.float32),
            pltpu.VMEM((_BPW * D,), jnp.float32),
            pltpu.VMEM((_CGRP + 1, D, 128), jnp.float32),
            pltpu.VMEM((_BPW,), jnp.float32),
            pltpu.SemaphoreType.DMA,
            pltpu.SemaphoreType.DMA,
        ],
    )(_content_kernel)
    return k2(content_sorted, comp_perm, content_tt, content_tail, u_emb)


def kernel(user, content, user_table, content_table):
    u_order = jnp.argsort(user)
    c_order = jnp.argsort(content)
    user_sorted = user[u_order]
    content_sorted = content[c_order]
    inv_u = jnp.zeros((B,), jnp.int32).at[u_order].set(
        jnp.arange(B, dtype=jnp.int32))
    comp = inv_u[c_order]
    out_sorted = _run(user_sorted, content_sorted, comp,
                      user_table.T, content_table.T,
                      user_table[NU - 128:, :].T,
                      content_table[NC_TAB - 128:, :].T)
    out = jnp.zeros((B,), jnp.float32).at[c_order].set(out_sorted)
    return out.reshape(B, 1)


# speedup vs baseline: 3.1808x; 1.7119x over previous
import functools

import jax
import jax.numpy as jnp
from jax import lax
from jax.experimental import pallas as pl
from jax.experimental.pallas import tpu as pltpu
from jax.experimental.pallas import tpu_sc as plsc

B = 16384
D = 64
NU = 1000000
NC_TAB = 100000

_info = plsc.get_sparse_core_info()
_NCORE, _NSUB = _info.num_cores, _info.num_subcores
_NW = _NCORE * _NSUB
_BPW = B // _NW
_FP = 16

_U_LAST = (NU - 1) // 128
_UGRP = 8


def _sread(ref, i):
    return ref[pl.ds(i, 16)][0]


def _stage_segments(r_v, seg_s):
    seg_s[0] = 0

    def body(p, ns):
        new = lax.ne(lax.shift_right_logical(_sread(r_v, _FP + p), 7),
                     lax.shift_right_logical(_sread(r_v, _FP + p - 1), 7))

        def write(n):
            seg_s[n] = p
            return n + 1

        return lax.cond(new, write, lambda n: n, ns)

    ns = lax.fori_loop(1, _BPW, body, 1)
    seg_s[ns] = _BPW
    return ns


def _user_kernel(skeys, user_raw, user_tt, user_tail, emb_out,
                 seg_s, uraw_v, key_v, r_v, i_v, slot_v, col_v, stage,
                 buf, slabsem, wsem):
    wid = lax.axis_index("s") * _NCORE + lax.axis_index("c")
    base = wid * _BPW
    lanes = lax.iota(jnp.int32, 16)

    pltpu.sync_copy(user_raw, uraw_v)
    pltpu.sync_copy(skeys.at[pl.ds(base, _BPW)], key_v.at[pl.ds(_FP, _BPW)])
    pltpu.sync_copy(user_tail, buf.at[_UGRP])

    def decode(k, c):
        kv = key_v[pl.ds(_FP + k * 16, 16)]
        iv = lax.bitwise_and(kv, (1 << 14) - 1)
        rv = plsc.load_gather(uraw_v, [iv])
        i_v[pl.ds(k * 16, 16)] = iv
        r_v[pl.ds(_FP + k * 16, 16)] = rv
        return c

    lax.fori_loop(0, _BPW // 16, decode, 0)
    first = r_v[pl.ds(_FP, 16)]
    r_v[pl.ds(0, 16)] = jnp.full((16,), 1, jnp.int32) * first[0]

    ns = _stage_segments(r_v, seg_s)

    lastv = jnp.full((16,), _U_LAST, jnp.int32)

    def slots(k, segc):
        v = r_v[pl.ds(_FP + k * 16, 16)]
        prev = r_v[pl.ds(_FP + k * 16 - 1, 16)]
        vs = lax.shift_right_logical(v, 7)
        b = (vs != lax.shift_right_logical(prev, 7)).astype(jnp.int32)
        segs = plsc.cumsum(b) + segc
        is_last = vs == lastv
        slot = jnp.where(is_last, jnp.int32(_UGRP),
                         lax.bitwise_and(segs, _UGRP - 1))
        col = jnp.where(is_last, v - (NU - 128), lax.bitwise_and(v, 127))
        slot_v[pl.ds(k * 16, 16)] = slot
        col_v[pl.ds(k * 16, 16)] = col
        return segs[15]

    lax.fori_loop(0, _BPW // 16, slots, jnp.int32(0))

    ngrp = (ns + _UGRP - 1) // _UGRP

    def group(g, carry):
        nf = jnp.int32(0)
        for j in range(_UGRP):
            s = g * _UGRP + j
            valid = s < ns
            st = seg_s[jnp.minimum(s, ns)]
            slab = lax.shift_right_logical(_sread(r_v, _FP + st), 7)
            fire = jnp.logical_and(valid, lax.ne(slab, _U_LAST))

            @pl.when(fire)
            def _():
                off = pl.multiple_of(slab * 128, 128)
                pltpu.async_copy(user_tt.at[:, pl.ds(off, 128)],
                                 buf.at[j], slabsem)

            nf = nf + fire.astype(jnp.int32)

        def drain(i, c):
            pltpu.make_async_copy(user_tt.at[:, pl.ds(0, 128)],
                                  buf.at[0], slabsem).wait()
            return c

        lax.fori_loop(0, nf, drain, 0)

        pstart = seg_s[jnp.minimum(g * _UGRP, ns)]
        pend = seg_s[jnp.minimum((g + 1) * _UGRP, ns)]

        def pblock(pb, c):
            pos = pb * 16 + lanes
            m = jnp.logical_and(pos >= pstart, pos < pend)
            sl = slot_v[pl.ds(pb * 16, 16)]
            co = col_v[pl.ds(pb * 16, 16)]
            wbase = pos * D
            for d in range(D):
                dv = jnp.full((16,), d, jnp.int32)
                val = plsc.load_gather(buf, [sl, dv, co])
                plsc.store_scatter(stage, [wbase + d], val, mask=m)
            return c

        lax.fori_loop(lax.shift_right_logical(pstart, 4),
                      lax.shift_right_logical(pend + 15, 4), pblock, 0)
        return carry

    lax.fori_loop(0, ngrp, group, 0)

    def emit(p, c):
        i = _sread(i_v, p)
        pltpu.async_copy(
            stage.at[pl.ds(pl.multiple_of(p * D, D), D)],
            emb_out.at[pl.ds(pl.multiple_of(i * D, D), D)], wsem)
        return c

    lax.fori_loop(0, _BPW, emit, 0)
    pltpu.make_async_copy(stage, emb_out.at[pl.ds(0, _BPW * D)],
                          wsem).wait()


def _content_kernel(content_idx, content_table, u_emb, out_hbm,
                    cidx_v, crows, urows, out_v, csem):
    wid = lax.axis_index("s") * _NCORE + lax.axis_index("c")
    base = wid * _BPW
    lanes = lax.iota(jnp.int32, 16)

    pltpu.sync_copy(content_idx.at[pl.ds(wid * 4, 4)], cidx_v)
    pltpu.sync_copy(u_emb.at[pl.ds(base * D, _BPW * D)], urows)

    copies = []
    for j in range(4):
        copies.append(pltpu.async_copy(
            content_table.at[cidx_v.at[j]],
            crows.at[pl.ds(j * 128, 128)], csem))
    for c in copies:
        c.wait()

    def dblk(h, c):
        prow = lanes + h * 16
        rows = prow * D
        acc0 = jnp.zeros((16,), jnp.float32)
        acc1 = jnp.zeros((16,), jnp.float32)
        for d in range(0, D, 2):
            d0 = jnp.full((16,), d, jnp.int32)
            d1 = jnp.full((16,), d + 1, jnp.int32)
            acc0 += (plsc.load_gather(urows, [rows + d])
                     * plsc.load_gather(crows, [prow, d0]))
            acc1 += (plsc.load_gather(urows, [rows + (d + 1)])
                     * plsc.load_gather(crows, [prow, d1]))
        out_v[pl.ds(h * 16, 16)] = acc0 + acc1
        return c

    lax.fori_loop(0, _BPW // 16, dblk, 0)

    pltpu.sync_copy(out_v, out_hbm.at[pl.ds(base, _BPW)])


@jax.jit
def _run(skeys, user, content_idx2d, user_tt, content_table, user_tail):
    mesh = plsc.VectorSubcoreMesh(core_axis_name="c", subcore_axis_name="s")

    k1 = functools.partial(
        pl.kernel, mesh=mesh,
        out_type=jax.ShapeDtypeStruct((B * D,), jnp.float32),
        compiler_params=pltpu.CompilerParams(needs_layout_passes=False),
        scratch_types=[
            pltpu.SMEM((_BPW + 1,), jnp.int32),
            pltpu.VMEM((B,), jnp.int32),
            pltpu.VMEM((_FP + _BPW + 16,), jnp.int32),
            pltpu.VMEM((_FP + _BPW + 16,), jnp.int32),
            pltpu.VMEM((_BPW + 16,), jnp.int32),
            pltpu.VMEM((_BPW + 16,), jnp.int32),
            pltpu.VMEM((_BPW + 16,), jnp.int32),
            pltpu.VMEM((_BPW * D,), jnp.float32),
            pltpu.VMEM((_UGRP + 1, D, 128), jnp.float32),
            pltpu.SemaphoreType.DMA,
            pltpu.SemaphoreType.DMA,
        ],
    )(_user_kernel)
    u_emb = k1(skeys, user, user_tt, user_tail)

    k2 = functools.partial(
        pl.kernel, mesh=mesh,
        out_type=jax.ShapeDtypeStruct((B,), jnp.float32),
        compiler_params=pltpu.CompilerParams(
            needs_layout_passes=False, use_tc_tiling_on_sc=False),
        scratch_types=[
            pltpu.VMEM((4, 128), jnp.int32),
            pltpu.VMEM((_BPW, D), jnp.float32),
            pltpu.VMEM((_BPW * D,), jnp.float32),
            pltpu.VMEM((_BPW,), jnp.float32),
            pltpu.SemaphoreType.DMA,
        ],
    )(_content_kernel)
    return k2(content_idx2d, content_table, u_emb)


def kernel(user, content, user_table, content_table):
    keys = lax.bitwise_or(
        lax.shift_left(lax.shift_right_logical(user, 7), 14),
        jnp.arange(B, dtype=jnp.int32))
    skeys = lax.sort(keys, is_stable=False)
    out = _run(skeys, user, content.reshape(B // 128, 128),
               user_table.T, content_table,
               user_table[NU - 128:, :].T)
    return out.reshape(B, 1)
